# Initial kernel scaffold; baseline (speedup 1.0000x reference)
import jax
import jax.numpy as jnp
from jax.experimental import pallas as pl


def kernel(z, codebook, W_down, W_up):
    raise NotImplementedError("write your pallas kernel here")


# speedup vs baseline: 1.1318x; 1.1318x over previous
import functools

import jax
import jax.numpy as jnp
from jax import lax
from jax.experimental import pallas as pl
from jax.experimental.pallas import tpu as pltpu
from jax.experimental.pallas import tpu_sc as plsc

B, T, D_IN = 16, 1024, 512
D_EMB = 256
K = 8192
M = B * T

M_TILE = 2048
K_TILE = 1024


def _down_body(z_ref, wd_ref, ze_ref):
    ze_ref[...] = lax.dot_general(
        z_ref[...], wd_ref[...], (((1,), (1,)), ((), ())))


def _down(z_flat, w_down):
    return pl.pallas_call(
        _down_body,
        grid=(M // M_TILE,),
        in_specs=[
            pl.BlockSpec((M_TILE, D_IN), lambda m: (m, 0)),
            pl.BlockSpec((D_EMB, D_IN), lambda m: (0, 0)),
        ],
        out_specs=pl.BlockSpec((M_TILE, D_EMB), lambda m: (m, 0)),
        out_shape=jax.ShapeDtypeStruct((M, D_EMB), jnp.float32),
        compiler_params=pltpu.CompilerParams(
            dimension_semantics=("arbitrary",)),
    )(z_flat, w_down)


def _assign_body(ze_ref, z2_ref, c2_ref, cb_ref, code_ref, minv_ref, mini_ref):
    k = pl.program_id(1)
    nk = pl.num_programs(1)

    a = ze_ref[...].astype(jnp.float32)
    p = lax.dot_general(a, cb_ref[...], (((1,), (1,)), ((), ())))
    dist = (z2_ref[...] - 2.0 * p) + c2_ref[0:1, :]
    lmin = jnp.min(dist, axis=1, keepdims=True)
    iota = lax.broadcasted_iota(jnp.int32, (M_TILE, K_TILE), 1) + k * K_TILE
    lidx = jnp.min(jnp.where(dist == lmin, iota, jnp.int32(2**30)),
                   axis=1, keepdims=True)

    @pl.when(k == 0)
    def _():
        minv_ref[...] = lmin
        mini_ref[...] = lidx

    @pl.when(k > 0)
    def _():
        better = lmin < minv_ref[...]
        mini_ref[...] = jnp.where(better, lidx, mini_ref[...])
        minv_ref[...] = jnp.minimum(lmin, minv_ref[...])

    @pl.when(k == nk - 1)
    def _():
        code_ref[...] = mini_ref[...]


def _assign(z_e_bf, z2, c2b, codebook):
    grid = (M // M_TILE, K // K_TILE)
    return pl.pallas_call(
        _assign_body,
        grid=grid,
        in_specs=[
            pl.BlockSpec((M_TILE, D_EMB), lambda m, k: (m, 0)),
            pl.BlockSpec((M_TILE, 1), lambda m, k: (m, 0)),
            pl.BlockSpec((8, K_TILE), lambda m, k: (0, k)),
            pl.BlockSpec((K_TILE, D_EMB), lambda m, k: (k, 0)),
        ],
        out_specs=pl.BlockSpec((M_TILE, 1), lambda m, k: (m, 0)),
        out_shape=jax.ShapeDtypeStruct((M, 1), jnp.int32),
        scratch_shapes=[
            pltpu.VMEM((M_TILE, 1), jnp.float32),
            pltpu.VMEM((M_TILE, 1), jnp.int32),
        ],
        compiler_params=pltpu.CompilerParams(
            dimension_semantics=("arbitrary", "arbitrary")),
    )(z_e_bf, z2, c2b, codebook)


def _sc_gather(codebook, idx):
    info = plsc.get_sparse_core_info()
    nw = info.num_cores * info.num_subcores
    b_per_w = M // nw
    chunk = 128
    n_chunks = b_per_w // chunk
    mesh = plsc.VectorSubcoreMesh(core_axis_name="c", subcore_axis_name="s")

    @functools.partial(
        pl.kernel, mesh=mesh,
        out_type=jax.ShapeDtypeStruct((M, D_EMB), jnp.float32),
        scratch_types=[
            pltpu.VMEM((n_chunks, chunk), jnp.int32),
            pltpu.VMEM((chunk, D_EMB), jnp.float32),
            pltpu.SemaphoreType.DMA,
        ],
    )
    def gather(table_hbm, idx_hbm, out_hbm, idx_v, rows_v, sem):
        wid = lax.axis_index("s") * info.num_cores + lax.axis_index("c")
        base = wid * b_per_w
        for c in range(n_chunks):
            pltpu.sync_copy(idx_hbm.at[pl.ds(base + c * chunk, chunk)],
                            idx_v.at[c])
            pltpu.async_copy(table_hbm.at[idx_v.at[c]], rows_v, sem).wait()
            pltpu.sync_copy(rows_v,
                            out_hbm.at[pl.ds(base + c * chunk, chunk)])

    return gather(codebook, idx)


def _up_body(zq_ref, ze_ref, wu_ref, out_ref, loss_ref):
    b = pl.program_id(0)
    zq = zq_ref[...]
    out_ref[...] = lax.dot_general(zq, wu_ref[...], (((1,), (1,)), ((), ())))
    d = zq - ze_ref[...]
    loss = jnp.sum(d * d) * (1.0 / (T * D_EMB))
    loss_ref[pl.ds(b, 1), :] = loss.reshape(1, 1)


def _up(z_q, z_e, w_up):
    return pl.pallas_call(
        _up_body,
        grid=(B,),
        in_specs=[
            pl.BlockSpec((T, D_EMB), lambda b: (b, 0)),
            pl.BlockSpec((T, D_EMB), lambda b: (b, 0)),
            pl.BlockSpec((D_IN, D_EMB), lambda b: (0, 0)),
        ],
        out_specs=[
            pl.BlockSpec((T, D_IN), lambda b: (b, 0)),
            pl.BlockSpec((B, 1), lambda b: (0, 0)),
        ],
        out_shape=[
            jax.ShapeDtypeStruct((M, D_IN), jnp.float32),
            jax.ShapeDtypeStruct((B, 1), jnp.float32),
        ],
        compiler_params=pltpu.CompilerParams(
            dimension_semantics=("arbitrary",)),
    )(z_q, z_e, w_up)


def kernel(z, codebook, W_down, W_up):
    z_flat = z.reshape(M, D_IN)
    z_e = _down(z_flat, W_down)
    z_e_bf = z_e.astype(jnp.bfloat16)
    z2 = jnp.sum(z_e ** 2, axis=1, keepdims=True)
    c2b = jnp.broadcast_to(
        jnp.sum(codebook ** 2, axis=1, keepdims=True).T, (8, K))
    code = _assign(z_e_bf, z2, c2b, codebook).reshape(M)
    z_q = _sc_gather(codebook, code)
    z_q_out, loss = _up(z_q, z_e, W_up)
    loss = loss.reshape(B)
    return (z_q_out.reshape(B, T, D_IN), loss, loss, code.reshape(B, T))


# speedup vs baseline: 1.1349x; 1.0028x over previous
import functools

import jax
import jax.numpy as jnp
from jax import lax
from jax.experimental import pallas as pl
from jax.experimental.pallas import tpu as pltpu
from jax.experimental.pallas import tpu_sc as plsc

B, T, D_IN = 16, 1024, 512
D_EMB = 256
K = 8192
M = B * T

M_TILE = 2048
K_TILE = 1024


def _down_body(z_ref, wd_ref, ze_ref):
    ze_ref[...] = lax.dot_general(
        z_ref[...], wd_ref[...], (((1,), (1,)), ((), ())))


def _down(z_flat, w_down):
    return pl.pallas_call(
        _down_body,
        grid=(M // M_TILE,),
        in_specs=[
            pl.BlockSpec((M_TILE, D_IN), lambda m: (m, 0)),
            pl.BlockSpec((D_EMB, D_IN), lambda m: (0, 0)),
        ],
        out_specs=pl.BlockSpec((M_TILE, D_EMB), lambda m: (m, 0)),
        out_shape=jax.ShapeDtypeStruct((M, D_EMB), jnp.float32),
        compiler_params=pltpu.CompilerParams(
            dimension_semantics=("arbitrary",)),
    )(z_flat, w_down)


def _assign_body(ze_ref, z2_ref, c2_ref, cb_ref, code_ref, minv_ref, mini_ref):
    k = pl.program_id(1)
    nk = pl.num_programs(1)

    p = lax.dot_general(ze_ref[...], cb_ref[...], (((1,), (1,)), ((), ())),
                        preferred_element_type=jnp.float32)
    dist = (z2_ref[...] - 2.0 * p) + c2_ref[0:1, :]
    lmin = jnp.min(dist, axis=1, keepdims=True)
    iota = lax.broadcasted_iota(jnp.int32, (M_TILE, K_TILE), 1) + k * K_TILE
    lidx = jnp.min(jnp.where(dist == lmin, iota, jnp.int32(2**30)),
                   axis=1, keepdims=True)

    @pl.when(k == 0)
    def _():
        minv_ref[...] = lmin
        mini_ref[...] = lidx

    @pl.when(k > 0)
    def _():
        better = lmin < minv_ref[...]
        mini_ref[...] = jnp.where(better, lidx, mini_ref[...])
        minv_ref[...] = jnp.minimum(lmin, minv_ref[...])

    @pl.when(k == nk - 1)
    def _():
        code_ref[...] = mini_ref[...]


def _assign(z_e_bf, z2, c2b, codebook):
    grid = (M // M_TILE, K // K_TILE)
    return pl.pallas_call(
        _assign_body,
        grid=grid,
        in_specs=[
            pl.BlockSpec((M_TILE, D_EMB), lambda m, k: (m, 0)),
            pl.BlockSpec((M_TILE, 1), lambda m, k: (m, 0)),
            pl.BlockSpec((8, K_TILE), lambda m, k: (0, k)),
            pl.BlockSpec((K_TILE, D_EMB), lambda m, k: (k, 0)),
        ],
        out_specs=pl.BlockSpec((M_TILE, 1), lambda m, k: (m, 0)),
        out_shape=jax.ShapeDtypeStruct((M, 1), jnp.int32),
        scratch_shapes=[
            pltpu.VMEM((M_TILE, 1), jnp.float32),
            pltpu.VMEM((M_TILE, 1), jnp.int32),
        ],
        compiler_params=pltpu.CompilerParams(
            dimension_semantics=("arbitrary", "arbitrary")),
    )(z_e_bf, z2, c2b, codebook)


def _sc_gather(codebook, idx):
    info = plsc.get_sparse_core_info()
    nw = info.num_cores * info.num_subcores
    b_per_w = M // nw
    chunk = 128
    n_chunks = b_per_w // chunk
    mesh = plsc.VectorSubcoreMesh(core_axis_name="c", subcore_axis_name="s")

    @functools.partial(
        pl.kernel, mesh=mesh,
        out_type=jax.ShapeDtypeStruct((M, D_EMB), jnp.float32),
        scratch_types=[
            pltpu.VMEM((n_chunks, chunk), jnp.int32),
            pltpu.VMEM((chunk, D_EMB), jnp.float32),
            pltpu.SemaphoreType.DMA,
        ],
    )
    def gather(table_hbm, idx_hbm, out_hbm, idx_v, rows_v, sem):
        wid = lax.axis_index("s") * info.num_cores + lax.axis_index("c")
        base = wid * b_per_w
        for c in range(n_chunks):
            pltpu.sync_copy(idx_hbm.at[pl.ds(base + c * chunk, chunk)],
                            idx_v.at[c])
            pltpu.async_copy(table_hbm.at[idx_v.at[c]], rows_v, sem).wait()
            pltpu.sync_copy(rows_v,
                            out_hbm.at[pl.ds(base + c * chunk, chunk)])

    return gather(codebook, idx)


def _up_body(zq_ref, ze_ref, wu_ref, out_ref, loss_ref):
    b = pl.program_id(0)
    zq = zq_ref[...]
    out_ref[...] = lax.dot_general(zq, wu_ref[...], (((1,), (1,)), ((), ())))
    d = zq - ze_ref[...]
    loss = jnp.sum(d * d) * (1.0 / (T * D_EMB))
    loss_ref[pl.ds(b, 1), :] = loss.reshape(1, 1)


def _up(z_q, z_e, w_up):
    return pl.pallas_call(
        _up_body,
        grid=(B,),
        in_specs=[
            pl.BlockSpec((T, D_EMB), lambda b: (b, 0)),
            pl.BlockSpec((T, D_EMB), lambda b: (b, 0)),
            pl.BlockSpec((D_IN, D_EMB), lambda b: (0, 0)),
        ],
        out_specs=[
            pl.BlockSpec((T, D_IN), lambda b: (b, 0)),
            pl.BlockSpec((B, 1), lambda b: (0, 0)),
        ],
        out_shape=[
            jax.ShapeDtypeStruct((M, D_IN), jnp.float32),
            jax.ShapeDtypeStruct((B, 1), jnp.float32),
        ],
        compiler_params=pltpu.CompilerParams(
            dimension_semantics=("arbitrary",)),
    )(z_q, z_e, w_up)


def kernel(z, codebook, W_down, W_up):
    z_flat = z.reshape(M, D_IN)
    z_e = _down(z_flat, W_down)
    z_e_bf = z_e.astype(jnp.bfloat16)
    z2 = jnp.sum(z_e ** 2, axis=1, keepdims=True)
    c2b = jnp.broadcast_to(
        jnp.sum(codebook ** 2, axis=1, keepdims=True).T, (8, K))
    code = _assign(z_e_bf, z2, c2b,
                   codebook.astype(jnp.bfloat16)).reshape(M)
    z_q = _sc_gather(codebook, code)
    z_q_out, loss = _up(z_q, z_e, W_up)
    loss = loss.reshape(B)
    return (z_q_out.reshape(B, T, D_IN), loss, loss, code.reshape(B, T))
